# baseline (speedup 1.0000x reference)
import jax
import jax.numpy as jnp
from jax.experimental import pallas as pl
from jax.experimental.pallas import tpu as pltpu

VOCAB = 100000
EMBED_DIM = 64
BATCH = 1024
HIDDEN = 128
VT = 4096
NV = (VOCAB + VT - 1) // VT


def _embed_kernel(idx_ref, table_ref, w1_ref, b1_ref, h_ref, e0, e1):
    def body(i, carry):
        e0[pl.ds(i, 1), :] = table_ref[pl.ds(idx_ref[i, 0], 1), :]
        e1[pl.ds(i, 1), :] = table_ref[pl.ds(idx_ref[i, 1], 1), :]
        return carry

    jax.lax.fori_loop(0, BATCH, body, 0)
    w1a = w1_ref[:, :EMBED_DIM].astype(jnp.bfloat16)
    w1b = w1_ref[:, EMBED_DIM:].astype(jnp.bfloat16)
    dn = (((1,), (1,)), ((), ()))
    acc = jax.lax.dot_general(e0[...].astype(jnp.bfloat16), w1a, dn,
                              preferred_element_type=jnp.float32)
    acc += jax.lax.dot_general(e1[...].astype(jnp.bfloat16), w1b, dn,
                               preferred_element_type=jnp.float32)
    h_ref[...] = jnp.maximum(acc + b1_ref[...], 0.0)


def _stats_kernel(h_ref, w2_ref, b2_ref, m_ref, s_ref):
    v = pl.program_id(0)

    @pl.when(v == 0)
    def _():
        m_ref[...] = jnp.full((BATCH, 1), -1e30, jnp.float32)
        s_ref[...] = jnp.zeros((BATCH, 1), jnp.float32)

    dn = (((1,), (1,)), ((), ()))
    logits = jax.lax.dot_general(
        h_ref[...].astype(jnp.bfloat16), w2_ref[...].astype(jnp.bfloat16),
        dn, preferred_element_type=jnp.float32) + b2_ref[...]
    col = jax.lax.broadcasted_iota(jnp.int32, (1, VT), 1) + v * VT
    logits = jnp.where(col < VOCAB, logits, -1e30)
    m_old = m_ref[...]
    m_new = jnp.maximum(m_old, jnp.max(logits, axis=1, keepdims=True))
    s_ref[...] = (s_ref[...] * jnp.exp(m_old - m_new)
                  + jnp.sum(jnp.exp(logits - m_new), axis=1, keepdims=True))
    m_ref[...] = m_new


def _out_kernel(h_ref, w2_ref, b2_ref, m_ref, s_ref, out_ref):
    dn = (((1,), (1,)), ((), ()))
    logits = jax.lax.dot_general(
        h_ref[...].astype(jnp.bfloat16), w2_ref[...].astype(jnp.bfloat16),
        dn, preferred_element_type=jnp.float32) + b2_ref[...]
    out_ref[...] = logits - (m_ref[...] + jnp.log(s_ref[...]))


def kernel(inputs, emb_table, W1, b1, W2, b2):
    b1r = b1.reshape(1, HIDDEN)
    b2r = b2.reshape(1, VOCAB)

    h = pl.pallas_call(
        _embed_kernel,
        out_shape=jax.ShapeDtypeStruct((BATCH, 2 * EMBED_DIM), jnp.float32),
        in_specs=[
            pl.BlockSpec(memory_space=pltpu.SMEM),
            pl.BlockSpec((VOCAB, EMBED_DIM), lambda: (0, 0)),
            pl.BlockSpec((HIDDEN, 2 * EMBED_DIM), lambda: (0, 0)),
            pl.BlockSpec((1, HIDDEN), lambda: (0, 0)),
        ],
        out_specs=pl.BlockSpec((BATCH, 2 * EMBED_DIM), lambda: (0, 0)),
        scratch_shapes=[
            pltpu.VMEM((BATCH, EMBED_DIM), jnp.float32),
            pltpu.VMEM((BATCH, EMBED_DIM), jnp.float32),
        ],
    )(inputs, emb_table, W1, b1r)

    m, s = pl.pallas_call(
        _stats_kernel,
        grid=(NV,),
        out_shape=(
            jax.ShapeDtypeStruct((BATCH, 1), jnp.float32),
            jax.ShapeDtypeStruct((BATCH, 1), jnp.float32),
        ),
        in_specs=[
            pl.BlockSpec((BATCH, 2 * EMBED_DIM), lambda v: (0, 0)),
            pl.BlockSpec((VT, HIDDEN), lambda v: (v, 0)),
            pl.BlockSpec((1, VT), lambda v: (0, v)),
        ],
        out_specs=(
            pl.BlockSpec((BATCH, 1), lambda v: (0, 0)),
            pl.BlockSpec((BATCH, 1), lambda v: (0, 0)),
        ),
    )(h, W2, b2r)

    out = pl.pallas_call(
        _out_kernel,
        grid=(NV,),
        out_shape=jax.ShapeDtypeStruct((BATCH, VOCAB), jnp.float32),
        in_specs=[
            pl.BlockSpec((BATCH, 2 * EMBED_DIM), lambda v: (0, 0)),
            pl.BlockSpec((VT, HIDDEN), lambda v: (v, 0)),
            pl.BlockSpec((1, VT), lambda v: (0, v)),
            pl.BlockSpec((BATCH, 1), lambda v: (0, 0)),
            pl.BlockSpec((BATCH, 1), lambda v: (0, 0)),
        ],
        out_specs=pl.BlockSpec((BATCH, VT), lambda v: (0, v)),
    )(h, W2, b2r, m, s)

    return out
